# baseline (speedup 1.0000x reference)
import jax
import jax.numpy as jnp
from jax.experimental import pallas as pl
from jax.experimental.pallas import tpu as pltpu

_N = 4096
_BR = 256
_NBLK = _N // _BR


def _berhu_body(x_ref, y_ref, out_ref, acc_ref):
    p = pl.program_id(0)
    j = pl.program_id(1)
    d = jnp.abs(x_ref[...] - y_ref[...])

    @pl.when(jnp.logical_and(p == 0, j == 0))
    def _():
        acc_ref[0] = 0.0

    @pl.when(p == 0)
    def _():
        acc_ref[0] = jnp.maximum(acc_ref[0], jnp.max(d))

    @pl.when(jnp.logical_and(p == 1, j == 0))
    def _():
        acc_ref[1] = 0.0
        acc_ref[2] = 0.0
        acc_ref[3] = 0.0

    @pl.when(p == 1)
    def _():
        c = jnp.maximum(acc_ref[0] * 0.2, 0.0001)
        large = d > c
        acc_ref[1] += jnp.sum(jnp.where(large, 0.0, d))
        acc_ref[2] += jnp.sum(jnp.where(large, d * d, 0.0))
        acc_ref[3] += jnp.sum(large.astype(jnp.float32))

    @pl.when(jnp.logical_and(p == 1, j == _NBLK - 1))
    def _():
        c = jnp.maximum(acc_ref[0] * 0.2, 0.0001)
        out_ref[0] = acc_ref[1] + (acc_ref[2] / c + c * acc_ref[3]) / 2.0


def kernel(prediction, target):
    x = prediction.reshape(_N, _N)
    y = target.reshape(_N, _N)
    spec = pl.BlockSpec((_BR, _N), lambda p, j: (j, 0))
    out = pl.pallas_call(
        _berhu_body,
        grid=(2, _NBLK),
        in_specs=[spec, spec],
        out_specs=pl.BlockSpec(memory_space=pltpu.SMEM),
        out_shape=jax.ShapeDtypeStruct((1,), jnp.float32),
        scratch_shapes=[pltpu.SMEM((4,), jnp.float32)],
        compiler_params=pltpu.CompilerParams(
            dimension_semantics=("arbitrary", "arbitrary"),
        ),
    )(x, y)
    return out.reshape(())


# speedup vs baseline: 3.9087x; 3.9087x over previous
import jax
import jax.numpy as jnp
from jax.experimental import pallas as pl
from jax.experimental.pallas import tpu as pltpu

_B = 64
_BB = 4
_NBLK = _B // _BB
_NTOT = 64 * 512 * 512


def _berhu_body(x_ref, y_ref, out_ref, acc_ref, cache_ref):
    p = pl.program_id(0)
    j = pl.program_id(1)

    @pl.when(p == 0)
    def _():
        d = jnp.abs(x_ref[...] - y_ref[...])

        @pl.when(j == 0)
        def _():
            acc_ref[0] = 0.0

        acc_ref[0] = jnp.maximum(acc_ref[0], jnp.max(d))
        cache_ref[pl.ds(j * _BB, _BB)] = d.astype(jnp.bfloat16)

    @pl.when(p == 1)
    def _():
        @pl.when(j == 0)
        def _():
            acc_ref[1] = 0.0
            acc_ref[2] = 0.0

        c = jnp.maximum(acc_ref[0] * 0.2, 0.0001)
        d = cache_ref[pl.ds(j * _BB, _BB)].astype(jnp.float32)
        f = jnp.maximum(d, c)
        acc_ref[1] += jnp.sum(jnp.minimum(d, c))
        acc_ref[2] += jnp.sum(f * f)

        @pl.when(j == _NBLK - 1)
        def _():
            out_ref[0] = (acc_ref[1] + acc_ref[2] / (2.0 * c)
                          - (0.5 * _NTOT) * c)


def kernel(prediction, target):
    spec = pl.BlockSpec(
        (_BB, 1, 512, 512), lambda p, j: (jnp.where(p == 0, j, 0), 0, 0, 0))
    out = pl.pallas_call(
        _berhu_body,
        grid=(2, _NBLK),
        in_specs=[spec, spec],
        out_specs=pl.BlockSpec(memory_space=pltpu.SMEM),
        out_shape=jax.ShapeDtypeStruct((1,), jnp.float32),
        scratch_shapes=[
            pltpu.SMEM((4,), jnp.float32),
            pltpu.VMEM((_B, 1, 512, 512), jnp.bfloat16),
        ],
        compiler_params=pltpu.CompilerParams(
            dimension_semantics=("arbitrary", "arbitrary"),
            vmem_limit_bytes=64 * 1024 * 1024,
        ),
    )(prediction, target)
    return out.reshape(())


# speedup vs baseline: 4.4921x; 1.1493x over previous
import jax
import jax.numpy as jnp
from jax.experimental import pallas as pl
from jax.experimental.pallas import tpu as pltpu

_B = 64
_BB = 4
_NBLK = _B // _BB
_NTOT = 64 * 512 * 512


def _berhu_body(x_ref, y_ref, out_ref, acc_ref, vacc_ref, cache_ref):
    p = pl.program_id(0)
    j = pl.program_id(1)

    @pl.when(p == 0)
    def _():
        d = jnp.abs(x_ref[...] - y_ref[...])

        @pl.when(j == 0)
        def _():
            acc_ref[0] = 0.0

        acc_ref[0] = jnp.maximum(acc_ref[0], jnp.max(d))
        cache_ref[pl.ds(j * _BB, _BB)] = d.astype(jnp.bfloat16)

    @pl.when(p == 1)
    def _():
        @pl.when(j == 0)
        def _():
            vacc_ref[...] = jnp.zeros_like(vacc_ref)

        cb = jnp.maximum(acc_ref[0] * 0.2, 0.0001).astype(jnp.bfloat16)
        d = cache_ref[pl.ds(j * _BB, _BB)].reshape(_BB * 512, 512)
        e = jnp.minimum(d, cb)
        f = jnp.maximum(d, cb)
        ones = jnp.ones((_BB * 512,), jnp.bfloat16)
        se = jax.lax.dot_general(ones, e, (((0,), (0,)), ((), ())),
                                 preferred_element_type=jnp.float32)
        sq = jax.lax.dot_general(ones, f * f, (((0,), (0,)), ((), ())),
                                 preferred_element_type=jnp.float32)
        vacc_ref[0, :] += se
        vacc_ref[1, :] += sq

        @pl.when(j == _NBLK - 1)
        def _():
            c32 = cb.astype(jnp.float32)
            out_ref[0] = (jnp.sum(vacc_ref[0, :])
                          + jnp.sum(vacc_ref[1, :]) / (2.0 * c32)
                          - (0.5 * _NTOT) * c32)


def kernel(prediction, target):
    spec = pl.BlockSpec(
        (_BB, 1, 512, 512), lambda p, j: (jnp.where(p == 0, j, 0), 0, 0, 0))
    out = pl.pallas_call(
        _berhu_body,
        grid=(2, _NBLK),
        in_specs=[spec, spec],
        out_specs=pl.BlockSpec(memory_space=pltpu.SMEM),
        out_shape=jax.ShapeDtypeStruct((1,), jnp.float32),
        scratch_shapes=[
            pltpu.SMEM((4,), jnp.float32),
            pltpu.VMEM((2, 512), jnp.float32),
            pltpu.VMEM((_B, 1, 512, 512), jnp.bfloat16),
        ],
        compiler_params=pltpu.CompilerParams(
            dimension_semantics=("arbitrary", "arbitrary"),
            vmem_limit_bytes=64 * 1024 * 1024,
        ),
    )(prediction, target)
    return out.reshape(())


# speedup vs baseline: 4.8377x; 1.0769x over previous
import jax
import jax.numpy as jnp
from jax.experimental import pallas as pl
from jax.experimental.pallas import tpu as pltpu

_B = 64
_BB = 4
_NBLK = _B // _BB


def _berhu_body(x_ref, y_ref, out_ref, acc_ref, vacc_ref, cache_ref):
    p = pl.program_id(0)
    j = pl.program_id(1)

    @pl.when(p == 0)
    def _():
        d = jnp.abs(x_ref[...] - y_ref[...])

        @pl.when(j == 0)
        def _():
            acc_ref[0] = 0.0
            acc_ref[1] = 0.0

        acc_ref[0] = jnp.maximum(acc_ref[0], jnp.max(d))
        acc_ref[1] += jnp.sum(d)
        cache_ref[pl.ds(j * _BB, _BB)] = d.astype(jnp.bfloat16)

    @pl.when(p == 1)
    def _():
        @pl.when(j == 0)
        def _():
            vacc_ref[...] = jnp.zeros_like(vacc_ref)

        cb = jnp.maximum(acc_ref[0] * 0.2, 0.0001).astype(jnp.bfloat16)
        d = cache_ref[pl.ds(j * _BB, _BB)].reshape(_BB * 512, 512)
        u = jnp.maximum(d - cb, jnp.bfloat16(0.0))
        ones = jnp.ones((_BB * 512,), jnp.bfloat16)
        sq = jax.lax.dot_general(ones, u * u, (((0,), (0,)), ((), ())),
                                 preferred_element_type=jnp.float32)
        vacc_ref[0, :] += sq

        @pl.when(j == _NBLK - 1)
        def _():
            c32 = cb.astype(jnp.float32)
            out_ref[0] = acc_ref[1] + jnp.sum(vacc_ref[0, :]) / (2.0 * c32)


def kernel(prediction, target):
    spec = pl.BlockSpec(
        (_BB, 1, 512, 512), lambda p, j: (jnp.where(p == 0, j, 0), 0, 0, 0))
    out = pl.pallas_call(
        _berhu_body,
        grid=(2, _NBLK),
        in_specs=[spec, spec],
        out_specs=pl.BlockSpec(memory_space=pltpu.SMEM),
        out_shape=jax.ShapeDtypeStruct((1,), jnp.float32),
        scratch_shapes=[
            pltpu.SMEM((4,), jnp.float32),
            pltpu.VMEM((1, 512), jnp.float32),
            pltpu.VMEM((_B, 1, 512, 512), jnp.bfloat16),
        ],
        compiler_params=pltpu.CompilerParams(
            dimension_semantics=("arbitrary", "arbitrary"),
            vmem_limit_bytes=64 * 1024 * 1024,
        ),
    )(prediction, target)
    return out.reshape(())
